# baseline (speedup 1.0000x reference)
import functools

import jax
import jax.numpy as jnp
from jax import lax
from jax.experimental import pallas as pl
from jax.experimental.pallas import tpu as pltpu
from jax.experimental.pallas import tpu_sc as plsc

E = 320000
V = 10000
D = 128
L = 16
NC = 2
NS = 16
NW = NC * NS
EPW = E // NW
C = 40
NCHUNK = EPW // C
DL = D // L

_mesh = plsc.VectorSubcoreMesh(core_axis_name="c", subcore_axis_name="s")


@functools.partial(
    pl.kernel,
    mesh=_mesh,
    out_type=[
        jax.ShapeDtypeStruct((NW, L), jnp.float32),
        jax.ShapeDtypeStruct((NW, L), jnp.float32),
    ],
    scratch_types=[
        pltpu.VMEM_SHARED((V, D), jnp.float32),
        pltpu.VMEM((EPW,), jnp.int32),
        pltpu.VMEM((EPW,), jnp.int32),
        pltpu.VMEM((C, D), jnp.float32),
        pltpu.VMEM((C, D), jnp.float32),
        pltpu.VMEM((C, D), jnp.float32),
        pltpu.VMEM((C, D), jnp.float32),
        pltpu.VMEM((L,), jnp.float32),
        pltpu.VMEM((L,), jnp.float32),
        pltpu.SemaphoreType.DMA,
        pltpu.SemaphoreType.DMA,
        pltpu.SemaphoreType.DMA,
        pltpu.SemaphoreType.DMA,
    ],
)
def _edge_partials(pred_hbm, src_hbm, dst_hbm, sum_out, cnt_out,
                   pred_sp, sidx, didx, srowsA, drowsA, srowsB, drowsB,
                   sum_v, cnt_v, semAS, semAH, semBS, semBH):
    sid = lax.axis_index("s")
    wid = sid * NC + lax.axis_index("c")
    base0 = wid * EPW
    zeros = jnp.zeros((L,), jnp.float32)

    vps = 632

    @pl.when(sid < NS - 1)
    def _copy_main():
        pltpu.sync_copy(pred_hbm.at[pl.ds(sid * vps, vps)],
                        pred_sp.at[pl.ds(sid * vps, vps)])

    @pl.when(sid == NS - 1)
    def _copy_tail():
        pltpu.sync_copy(pred_hbm.at[pl.ds((NS - 1) * vps, V - (NS - 1) * vps)],
                        pred_sp.at[pl.ds((NS - 1) * vps, V - (NS - 1) * vps)])

    pltpu.sync_copy(src_hbm.at[pl.ds(base0, EPW)], sidx)
    pltpu.sync_copy(dst_hbm.at[pl.ds(base0, EPW)], didx)
    plsc.subcore_barrier()

    def cnt_body(k, c):
        s = sidx[pl.ds(k * L, L)]
        d = didx[pl.ds(k * L, L)]
        m = (s != 0) | (d != 0)
        return c + jnp.where(m, 1.0, 0.0)

    cnt = lax.fori_loop(0, EPW // L, cnt_body, zeros)

    def issue(chunk, srows, drows, semS, semH):
        pltpu.async_copy(pred_sp.at[sidx.at[pl.ds(chunk * C, C)]], srows, semS)
        pltpu.async_copy(pred_hbm.at[didx.at[pl.ds(chunk * C, C)]], drows, semH)

    def drain(srows, drows, semS, semH):
        pltpu.make_async_copy(pred_sp.at[pl.ds(0, C)], srows, semS).wait()
        pltpu.make_async_copy(pred_hbm.at[pl.ds(0, C)], drows, semH).wait()

    def accum(srows, drows, accs):
        def edge_body(e, accs):
            new = []
            for j in range(DL):
                a = srows[e, pl.ds(j * L, L)]
                b = drows[e, pl.ds(j * L, L)]
                diff = a - b
                new.append(accs[j] + diff * diff)
            return tuple(new)
        return lax.fori_loop(0, C, edge_body, accs)

    issue(0, srowsA, drowsA, semAS, semAH)

    def pair_body(g, accs):
        issue(2 * g + 1, srowsB, drowsB, semBS, semBH)
        drain(srowsA, drowsA, semAS, semAH)
        accs = accum(srowsA, drowsA, accs)

        @pl.when(2 * g + 2 < NCHUNK)
        def _issue_next():
            issue(2 * g + 2, srowsA, drowsA, semAS, semAH)

        drain(srowsB, drowsB, semBS, semBH)
        return accum(srowsB, drowsB, accs)

    accs = lax.fori_loop(0, NCHUNK // 2, pair_body,
                         tuple(zeros for _ in range(DL)))

    tot = accs[0]
    for j in range(1, DL):
        tot = tot + accs[j]
    sum_v[...] = tot
    cnt_v[...] = cnt
    pltpu.sync_copy(sum_v, sum_out.at[wid])
    pltpu.sync_copy(cnt_v, cnt_out.at[wid])


def _finalize_body(sums_ref, cnts_ref, out_ref):
    out_ref[0, 0] = jnp.sum(sums_ref[...]) / jnp.sum(cnts_ref[...])


_finalize = pl.pallas_call(
    _finalize_body,
    out_shape=jax.ShapeDtypeStruct((1, 1), jnp.float32),
    in_specs=[
        pl.BlockSpec(memory_space=pltpu.VMEM),
        pl.BlockSpec(memory_space=pltpu.VMEM),
    ],
    out_specs=pl.BlockSpec(memory_space=pltpu.SMEM),
)


def kernel(pred, edge_list):
    src = edge_list[0]
    dst = edge_list[1]
    sums, cnts = _edge_partials(pred, src, dst)
    return _finalize(sums, cnts)[0, 0]


# speedup vs baseline: 1.0112x; 1.0112x over previous
import functools

import jax
import jax.numpy as jnp
from jax import lax
from jax.experimental import pallas as pl
from jax.experimental.pallas import tpu as pltpu
from jax.experimental.pallas import tpu_sc as plsc

E = 320000
V = 10000
D = 128
L = 16
NC = 2
NS = 16
NW = NC * NS
EPW = E // NW
C = 40
NCHUNK = EPW // C
DL = D // L

_mesh = plsc.VectorSubcoreMesh(core_axis_name="c", subcore_axis_name="s")


@functools.partial(
    pl.kernel,
    mesh=_mesh,
    out_type=[
        jax.ShapeDtypeStruct((NW, L), jnp.float32),
        jax.ShapeDtypeStruct((NW, L), jnp.float32),
    ],
    scratch_types=[
        pltpu.VMEM_SHARED((V, D), jnp.float32),
        pltpu.VMEM((EPW,), jnp.int32),
        pltpu.VMEM((EPW,), jnp.int32),
        pltpu.VMEM((C, D), jnp.float32),
        pltpu.VMEM((C, D), jnp.float32),
        pltpu.VMEM((C, D), jnp.float32),
        pltpu.VMEM((C, D), jnp.float32),
        pltpu.VMEM((L,), jnp.float32),
        pltpu.VMEM((L,), jnp.float32),
        pltpu.SemaphoreType.DMA,
        pltpu.SemaphoreType.DMA,
        pltpu.SemaphoreType.DMA,
        pltpu.SemaphoreType.DMA,
    ],
)
def _edge_partials(pred_hbm, src_hbm, dst_hbm, sum_out, cnt_out,
                   pred_sp, sidx, didx, srowsA, drowsA, srowsB, drowsB,
                   sum_v, cnt_v, semAS, semAH, semBS, semBH):
    sid = lax.axis_index("s")
    wid = sid * NC + lax.axis_index("c")
    base0 = wid * EPW
    zeros = jnp.zeros((L,), jnp.float32)

    vps = 632

    @pl.when(sid < NS - 1)
    def _copy_main():
        pltpu.sync_copy(pred_hbm.at[pl.ds(sid * vps, vps)],
                        pred_sp.at[pl.ds(sid * vps, vps)])

    @pl.when(sid == NS - 1)
    def _copy_tail():
        pltpu.sync_copy(pred_hbm.at[pl.ds((NS - 1) * vps, V - (NS - 1) * vps)],
                        pred_sp.at[pl.ds((NS - 1) * vps, V - (NS - 1) * vps)])

    pltpu.sync_copy(src_hbm.at[pl.ds(base0, EPW)], sidx)
    pltpu.sync_copy(dst_hbm.at[pl.ds(base0, EPW)], didx)
    plsc.subcore_barrier()

    def cnt_body(k, c):
        s = sidx[pl.ds(k * L, L)]
        d = didx[pl.ds(k * L, L)]
        m = (s != 0) | (d != 0)
        return c + jnp.where(m, 1.0, 0.0)

    cnt = lax.fori_loop(0, EPW // L, cnt_body, zeros)

    def issue(chunk, srows, drows, semS, semH):
        sl = sidx.at[pl.ds(chunk * C, C)]
        dl = didx.at[pl.ds(chunk * C, C)]

        @pl.when(chunk % 3 != 2)
        def _from_spmem():
            pltpu.async_copy(pred_sp.at[sl], srows, semS)
            pltpu.async_copy(pred_sp.at[dl], drows, semS)

        @pl.when(chunk % 3 == 2)
        def _from_hbm():
            pltpu.async_copy(pred_hbm.at[sl], srows, semH)
            pltpu.async_copy(pred_hbm.at[dl], drows, semH)

    def drain(chunk, srows, drows, semS, semH):
        @pl.when(chunk % 3 != 2)
        def _from_spmem():
            pltpu.make_async_copy(pred_sp.at[pl.ds(0, C)], srows, semS).wait()
            pltpu.make_async_copy(pred_sp.at[pl.ds(0, C)], drows, semS).wait()

        @pl.when(chunk % 3 == 2)
        def _from_hbm():
            pltpu.make_async_copy(pred_hbm.at[pl.ds(0, C)], srows, semH).wait()
            pltpu.make_async_copy(pred_hbm.at[pl.ds(0, C)], drows, semH).wait()

    def accum(srows, drows, accs):
        def edge_body(e, accs):
            new = []
            for j in range(DL):
                a = srows[e, pl.ds(j * L, L)]
                b = drows[e, pl.ds(j * L, L)]
                diff = a - b
                new.append(accs[j] + diff * diff)
            return tuple(new)
        return lax.fori_loop(0, C, edge_body, accs)

    issue(0, srowsA, drowsA, semAS, semAH)

    def pair_body(g, accs):
        issue(2 * g + 1, srowsB, drowsB, semBS, semBH)
        drain(2 * g, srowsA, drowsA, semAS, semAH)
        accs = accum(srowsA, drowsA, accs)

        @pl.when(2 * g + 2 < NCHUNK)
        def _issue_next():
            issue(2 * g + 2, srowsA, drowsA, semAS, semAH)

        drain(2 * g + 1, srowsB, drowsB, semBS, semBH)
        return accum(srowsB, drowsB, accs)

    accs = lax.fori_loop(0, NCHUNK // 2, pair_body,
                         tuple(zeros for _ in range(DL)))

    tot = accs[0]
    for j in range(1, DL):
        tot = tot + accs[j]
    sum_v[...] = tot
    cnt_v[...] = cnt
    pltpu.sync_copy(sum_v, sum_out.at[wid])
    pltpu.sync_copy(cnt_v, cnt_out.at[wid])


def _finalize_body(sums_ref, cnts_ref, out_ref):
    out_ref[0, 0] = jnp.sum(sums_ref[...]) / jnp.sum(cnts_ref[...])


_finalize = pl.pallas_call(
    _finalize_body,
    out_shape=jax.ShapeDtypeStruct((1, 1), jnp.float32),
    in_specs=[
        pl.BlockSpec(memory_space=pltpu.VMEM),
        pl.BlockSpec(memory_space=pltpu.VMEM),
    ],
    out_specs=pl.BlockSpec(memory_space=pltpu.SMEM),
)


def kernel(pred, edge_list):
    src = edge_list[0]
    dst = edge_list[1]
    sums, cnts = _edge_partials(pred, src, dst)
    return _finalize(sums, cnts)[0, 0]


# speedup vs baseline: 1.2552x; 1.2413x over previous
import functools

import jax
import jax.numpy as jnp
from jax import lax
from jax.experimental import pallas as pl
from jax.experimental.pallas import tpu as pltpu
from jax.experimental.pallas import tpu_sc as plsc

E = 320000
V = 10000
D = 128
L = 16
NC = 2
NS = 16
NW = NC * NS
EPW = E // NW
CX = 56
CH = 24
CI = CX + CH
IB = 2000
NB = EPW // IB
NIT = IB // CI
DL = D // L

_mesh = plsc.VectorSubcoreMesh(core_axis_name="c", subcore_axis_name="s")


@functools.partial(
    pl.kernel,
    mesh=_mesh,
    out_type=[
        jax.ShapeDtypeStruct((NW, L), jnp.float32),
        jax.ShapeDtypeStruct((NW, L), jnp.float32),
    ],
    scratch_types=[
        pltpu.VMEM_SHARED((V, D), jnp.float32),
        pltpu.VMEM((IB,), jnp.int32),
        pltpu.VMEM((IB,), jnp.int32),
        pltpu.VMEM((CX, D), jnp.float32),
        pltpu.VMEM((CX, D), jnp.float32),
        pltpu.VMEM((CX, D), jnp.float32),
        pltpu.VMEM((CX, D), jnp.float32),
        pltpu.VMEM((CH, D), jnp.float32),
        pltpu.VMEM((CH, D), jnp.float32),
        pltpu.VMEM((CH, D), jnp.float32),
        pltpu.VMEM((CH, D), jnp.float32),
        pltpu.VMEM((L,), jnp.float32),
        pltpu.VMEM((L,), jnp.float32),
        pltpu.SemaphoreType.DMA,
        pltpu.SemaphoreType.DMA,
        pltpu.SemaphoreType.DMA,
        pltpu.SemaphoreType.DMA,
    ],
)
def _edge_partials(pred_hbm, src_hbm, dst_hbm, sum_out, cnt_out,
                   pred_sp, sidx, didx,
                   sxA, dxA, sxB, dxB, shA, dhA, shB, dhB,
                   sum_v, cnt_v, semXA, semXB, semHA, semHB):
    sid = lax.axis_index("s")
    wid = sid * NC + lax.axis_index("c")
    base0 = wid * EPW
    zeros = jnp.zeros((L,), jnp.float32)

    vps = 632

    @pl.when(sid < NS - 1)
    def _copy_main():
        pltpu.sync_copy(pred_hbm.at[pl.ds(sid * vps, vps)],
                        pred_sp.at[pl.ds(sid * vps, vps)])

    @pl.when(sid == NS - 1)
    def _copy_tail():
        pltpu.sync_copy(pred_hbm.at[pl.ds((NS - 1) * vps, V - (NS - 1) * vps)],
                        pred_sp.at[pl.ds((NS - 1) * vps, V - (NS - 1) * vps)])

    plsc.subcore_barrier()

    def issue(it, sx, dx, sh, dh, semX, semH):
        bx = it * CI
        bh = it * CI + CX
        pltpu.async_copy(pred_sp.at[sidx.at[pl.ds(bx, CX)]], sx, semX)
        pltpu.async_copy(pred_sp.at[didx.at[pl.ds(bx, CX)]], dx, semX)
        pltpu.async_copy(pred_hbm.at[sidx.at[pl.ds(bh, CH)]], sh, semH)
        pltpu.async_copy(pred_hbm.at[didx.at[pl.ds(bh, CH)]], dh, semH)

    def drain(sx, dx, sh, dh, semX, semH):
        pltpu.make_async_copy(pred_sp.at[pl.ds(0, CX)], sx, semX).wait()
        pltpu.make_async_copy(pred_sp.at[pl.ds(0, CX)], dx, semX).wait()
        pltpu.make_async_copy(pred_hbm.at[pl.ds(0, CH)], sh, semH).wait()
        pltpu.make_async_copy(pred_hbm.at[pl.ds(0, CH)], dh, semH).wait()

    def accum(rows_s, rows_d, n, accs):
        def edge_body(e, accs):
            new = []
            for j in range(DL):
                a = rows_s[e, pl.ds(j * L, L)]
                b = rows_d[e, pl.ds(j * L, L)]
                diff = a - b
                new.append(accs[j] + diff * diff)
            return tuple(new)
        return lax.fori_loop(0, n, edge_body, accs)

    def accum_both(sx, dx, sh, dh, accs):
        accs = accum(sx, dx, CX, accs)
        return accum(sh, dh, CH, accs)

    accs = tuple(zeros for _ in range(DL))
    cnt = zeros

    for b in range(NB):
        base = base0 + b * IB
        pltpu.sync_copy(src_hbm.at[pl.ds(base, IB)], sidx)
        pltpu.sync_copy(dst_hbm.at[pl.ds(base, IB)], didx)

        def cnt_body(k, c):
            s = sidx[pl.ds(k * L, L)]
            d = didx[pl.ds(k * L, L)]
            m = (s != 0) | (d != 0)
            return c + jnp.where(m, 1.0, 0.0)

        cnt = lax.fori_loop(0, IB // L, cnt_body, cnt)

        issue(0, sxA, dxA, shA, dhA, semXA, semHA)

        def pair_body(g, accs):
            issue(2 * g + 1, sxB, dxB, shB, dhB, semXB, semHB)
            drain(sxA, dxA, shA, dhA, semXA, semHA)
            accs = accum_both(sxA, dxA, shA, dhA, accs)
            issue(2 * g + 2, sxA, dxA, shA, dhA, semXA, semHA)
            drain(sxB, dxB, shB, dhB, semXB, semHB)
            return accum_both(sxB, dxB, shB, dhB, accs)

        accs = lax.fori_loop(0, NIT // 2, pair_body, accs)
        drain(sxA, dxA, shA, dhA, semXA, semHA)
        accs = accum_both(sxA, dxA, shA, dhA, accs)

    tot = accs[0]
    for j in range(1, DL):
        tot = tot + accs[j]
    sum_v[...] = tot
    cnt_v[...] = cnt
    pltpu.sync_copy(sum_v, sum_out.at[wid])
    pltpu.sync_copy(cnt_v, cnt_out.at[wid])


def _finalize_body(sums_ref, cnts_ref, out_ref):
    out_ref[0, 0] = jnp.sum(sums_ref[...]) / jnp.sum(cnts_ref[...])


_finalize = pl.pallas_call(
    _finalize_body,
    out_shape=jax.ShapeDtypeStruct((1, 1), jnp.float32),
    in_specs=[
        pl.BlockSpec(memory_space=pltpu.VMEM),
        pl.BlockSpec(memory_space=pltpu.VMEM),
    ],
    out_specs=pl.BlockSpec(memory_space=pltpu.SMEM),
)


def kernel(pred, edge_list):
    src = edge_list[0]
    dst = edge_list[1]
    sums, cnts = _edge_partials(pred, src, dst)
    return _finalize(sums, cnts)[0, 0]
